# baseline (speedup 1.0000x reference)
import functools
import math

import jax
import jax.numpy as jnp
from jax import lax
from jax.experimental import pallas as pl
from jax.experimental.pallas import tpu as pltpu
from jax.experimental.pallas import tpu_sc as plsc

N_PART = 100000
N_EDGE = 6400000
SUPPORT = 0.05
KGRAD = -20.0 * 7.0 / (math.pi * SUPPORT * SUPPORT) / SUPPORT

NC, NS, L = 2, 16, 16
NW = NC * NS
B = 128
NP = 100352
RPT = NP // NS
LASTR = N_PART - 15 * RPT
EPW = N_EDGE // NW
CHUNK = 800
NCHUNK = EPW // CHUNK
NPAIR = NCHUNK // 2
CSTEPS = CHUNK // L
DBLKS = CHUNK // B + 2
DSLAB = DBLKS * 2 * B
UPC = 7 * B
UPW = 2 * UPC
NUPC = NP // UPC
VRP = 784
VRL = LASTR - 7 * VRP

_mesh = plsc.VectorSubcoreMesh(
    core_axis_name="c", subcore_axis_name="s", num_cores=NC, num_subcores=NS
)


@functools.partial(
    pl.kernel,
    out_type=jax.ShapeDtypeStruct((NC, NP), jnp.float32),
    mesh=_mesh,
    scratch_types=[
        pltpu.VMEM_SHARED((NP,), jnp.float32),
        pltpu.VMEM_SHARED((NP,), jnp.float32),
        pltpu.VMEM((NP,), jnp.int32),
        pltpu.VMEM((UPW,), jnp.float32),
        pltpu.VMEM((UPW,), jnp.float32),
        pltpu.VMEM((VRP,), jnp.float32),
        [pltpu.VMEM((CHUNK,), jnp.int32) for _ in range(2)],
        [pltpu.VMEM((CHUNK,), jnp.int32) for _ in range(2)],
        [pltpu.VMEM((CHUNK,), jnp.float32) for _ in range(2)],
        [pltpu.VMEM((DSLAB,), jnp.float32) for _ in range(2)],
        [pltpu.VMEM((CHUNK,), jnp.float32) for _ in range(2)],
        [pltpu.VMEM((CHUNK,), jnp.float32) for _ in range(2)],
        [pltpu.SemaphoreType.DMA for _ in range(2)],
        [pltpu.SemaphoreType.DMA for _ in range(2)],
        [pltpu.SemaphoreType.DMA for _ in range(2)],
        [pltpu.SemaphoreType.DMA for _ in range(2)],
    ],
    compiler_params=pltpu.CompilerParams(
        needs_layout_passes=False, use_tc_tiling_on_sc=False
    ),
)
def _sc_dpdt(ei_hbm, ej_hbm, v_hbm, rho_hbm, ublk_hbm, dblk_hbm, rad_hbm,
             out_hbm,
             vrt, dpdt, upk, pbufa, pbufb, vrbuf,
             ei, ej, rad, dst, vrj, cbuf, sbld, slin, sgat, ssc):
    cid = lax.axis_index("c")
    sid = lax.axis_index("s")
    wid = sid * NC + cid
    iota = lax.iota(jnp.int32, L)
    half = jnp.full((L,), 0x8000, jnp.int32)
    himask = jnp.full((L,), -0x10000, jnp.int32)

    def upack(p, pbuf):
        def body(k, inner):
            r = k * L + iota
            idx = ((r >> 7) << 8) + (r & (B - 1))
            bx = plsc.bitcast(plsc.load_gather(pbuf, [idx]), jnp.int32)
            by = plsc.bitcast(plsc.load_gather(pbuf, [idx + B]), jnp.int32)
            lo = lax.shift_right_logical(bx + half, 16)
            hi = (by + half) & himask
            upk[pl.ds(p * UPC + k * L, L)] = lo | hi
            return inner

        lax.fori_loop(0, UPC // L, body, 0)

    pltpu.async_copy(ublk_hbm.at[pl.ds(0, UPW)], pbufa, sbld[0])

    def upair(p2, carry):
        p = 2 * p2
        pltpu.make_async_copy(ublk_hbm.at[pl.ds(0, UPW)], pbufa, sbld[0]).wait()
        pltpu.async_copy(ublk_hbm.at[pl.ds((p + 1) * UPW, UPW)], pbufb, sbld[1])
        upack(p, pbufa)
        pltpu.make_async_copy(ublk_hbm.at[pl.ds(0, UPW)], pbufb, sbld[1]).wait()

        @pl.when(p2 < NUPC // 2 - 1)
        def _():
            pltpu.async_copy(ublk_hbm.at[pl.ds((p + 2) * UPW, UPW)], pbufa,
                             sbld[0])

        upack(p + 1, pbufb)
        return carry

    lax.fori_loop(0, NUPC // 2, upair, 0)

    nbase = sid * RPT

    def vrpiece(rows0, nrows):
        pltpu.sync_copy(v_hbm.at[pl.ds(rows0, nrows)], pbufa.at[pl.ds(0, nrows)])
        pltpu.sync_copy(rho_hbm.at[pl.ds(rows0, nrows)],
                        pbufa.at[pl.ds(VRP, nrows)])

        def vr_step(k, carry):
            vrbuf[pl.ds(k * L, L)] = (pbufa[pl.ds(k * L, L)]
                                      * pbufa[pl.ds(VRP + k * L, L)])
            return carry

        lax.fori_loop(0, nrows // L, vr_step, 0)
        pltpu.sync_copy(vrbuf.at[pl.ds(0, nrows)], vrt.at[pl.ds(rows0, nrows)])

    def vr_full(p, carry):
        vrpiece(nbase + p * VRP, VRP)
        return carry

    @pl.when(sid < NS - 1)
    def _():
        lax.fori_loop(0, 8, vr_full, 0)

    @pl.when(sid == NS - 1)
    def _():
        lax.fori_loop(0, 7, vr_full, 0)
        vrpiece(nbase + 7 * VRP, VRL)

    def zstep(k, carry):
        vrbuf[pl.ds(k * L, L)] = jnp.zeros((L,), jnp.float32)
        return carry

    lax.fori_loop(0, VRP // L, zstep, 0)

    def zpiece(p, carry):
        pltpu.sync_copy(vrbuf, dpdt.at[pl.ds(nbase + p * VRP, VRP)])
        return carry

    lax.fori_loop(0, 8, zpiece, 0)

    plsc.subcore_barrier()

    ebase0 = wid * EPW

    def slab_start(eb):
        return jnp.minimum(eb >> 7, N_EDGE // B - DBLKS)

    def issue_lin(eb, b):
        pltpu.async_copy(ei_hbm.at[pl.ds(eb, CHUNK)], ei[b], slin[b])
        pltpu.async_copy(ej_hbm.at[pl.ds(eb, CHUNK)], ej[b], slin[b])
        pltpu.async_copy(rad_hbm.at[pl.ds(eb, CHUNK)], rad[b], slin[b])
        pltpu.async_copy(dblk_hbm.at[pl.ds(slab_start(eb) * (2 * B), DSLAB)],
                         dst[b], slin[b])

    def wait_lin(b):
        pltpu.make_async_copy(ei_hbm.at[pl.ds(0, CHUNK)], ei[b], slin[b]).wait()
        pltpu.make_async_copy(ej_hbm.at[pl.ds(0, CHUNK)], ej[b], slin[b]).wait()
        pltpu.make_async_copy(rad_hbm.at[pl.ds(0, CHUNK)], rad[b],
                              slin[b]).wait()
        pltpu.make_async_copy(dblk_hbm.at[pl.ds(0, DSLAB)], dst[b],
                              slin[b]).wait()

    def compute(eb, b, k0, ksteps):
        off0 = eb - slab_start(eb) * B

        def step(k, inner):
            sl = pl.ds(k * L, L)
            r = rad[b][sl]
            q = jnp.minimum(jnp.maximum(r, 0.0), 1.0)
            om = 1.0 - q
            w = (om * om) * (om * q) * KGRAD
            off = off0 + k * L + iota
            idx = ((off >> 7) << 8) + (off & (B - 1))
            dx = plsc.load_gather(dst[b], [idx])
            dy = plsc.load_gather(dst[b], [idx + B])
            wj = plsc.load_gather(upk, [ej[b][sl]])
            wi = plsc.load_gather(upk, [ei[b][sl]])
            uxj = plsc.bitcast(wj << 16, jnp.float32)
            uyj = plsc.bitcast(wj & himask, jnp.float32)
            uxi = plsc.bitcast(wi << 16, jnp.float32)
            uyi = plsc.bitcast(wi & himask, jnp.float32)
            prod = (uxj - uxi) * dx + (uyj - uyi) * dy
            cbuf[b][sl] = prod * w * vrj[b][sl]
            return inner

        lax.fori_loop(k0, k0 + ksteps, step, 0)

    def wait_scatter(b):
        pltpu.make_async_copy(cbuf[b], dpdt.at[ei[b]], ssc[b]).wait()

    issue_lin(ebase0, 0)
    CH2 = CHUNK // 2

    def half(eb, b, skip_scwait, issue_next):
        wait_lin(b)
        ga = pltpu.async_copy(vrt.at[ej[b].at[pl.ds(0, CH2)]],
                              vrj[b].at[pl.ds(0, CH2)], sgat[b])
        gb = pltpu.async_copy(vrt.at[ej[b].at[pl.ds(CH2, CH2)]],
                              vrj[b].at[pl.ds(CH2, CH2)], sgat[b])

        @pl.when(jnp.logical_not(skip_scwait))
        def _():
            wait_scatter(1 - b)

        @pl.when(issue_next)
        def _():
            issue_lin(eb + CHUNK, 1 - b)

        ga.wait()
        compute(eb, b, 0, CSTEPS // 2)
        gb.wait()
        compute(eb, b, CSTEPS // 2, CSTEPS // 2)
        pltpu.async_copy(cbuf[b], dpdt.at[ei[b]], ssc[b], add=True)

    true_s = jnp.bool_(True)

    def pair_body(c2, carry):
        eb0 = ebase0 + (2 * c2) * CHUNK
        half(eb0, 0, c2 == 0, true_s)
        half(eb0 + CHUNK, 1, jnp.bool_(False), c2 < NPAIR - 1)
        return carry

    lax.fori_loop(0, NPAIR, pair_body, 0)
    wait_scatter(1)

    plsc.subcore_barrier()
    pltpu.sync_copy(dpdt.at[pl.ds(nbase, RPT)],
                    out_hbm.at[cid, pl.ds(nbase, RPT)])


def _combine_body(p_ref, o_ref):
    o_ref[...] = -(p_ref[0, :N_PART] + p_ref[1, :N_PART])


_combine = pl.pallas_call(
    _combine_body,
    out_shape=jax.ShapeDtypeStruct((N_PART,), jnp.float32),
)


def _planar_flat(x, nblocks):
    return x.reshape(nblocks, B, 2).transpose(0, 2, 1).reshape(-1)


def kernel(edge_i, edge_j, V, rho, u, distances, radialDistances):
    ei = edge_i.astype(jnp.int32)
    ej = edge_j.astype(jnp.int32)
    u_p = jnp.pad(u, ((0, NP - N_PART), (0, 0)))
    ublk = _planar_flat(u_p, NP // B)
    dblk = _planar_flat(distances, N_EDGE // B)
    part = _sc_dpdt(ei, ej, V, rho, ublk, dblk, radialDistances)
    return _combine(part)


# speedup vs baseline: 1.2078x; 1.2078x over previous
import functools
import math

import jax
import jax.numpy as jnp
from jax import lax
from jax.experimental import pallas as pl
from jax.experimental.pallas import tpu as pltpu
from jax.experimental.pallas import tpu_sc as plsc

N_PART = 100000
N_EDGE = 6400000
SUPPORT = 0.05
KGRAD = -20.0 * 7.0 / (math.pi * SUPPORT * SUPPORT) / SUPPORT

NC, NS, L = 2, 16, 16
NW = NC * NS
B = 128
NP = 100352
RPT = NP // NS
LASTR = N_PART - 15 * RPT
EPW = N_EDGE // NW
CHUNK = 800
NCHUNK = EPW // CHUNK
NPAIR = NCHUNK // 2
CSTEPS = CHUNK // L
DBLKS = CHUNK // B + 2
DSLAB = DBLKS * 2 * B
UPC = 7 * B
UPW = 2 * UPC
NUPC = NP // UPC
VRP = 784
VRL = LASTR - 7 * VRP

_mesh = plsc.VectorSubcoreMesh(
    core_axis_name="c", subcore_axis_name="s", num_cores=NC, num_subcores=NS
)


@functools.partial(
    pl.kernel,
    out_type=jax.ShapeDtypeStruct((NC, NP), jnp.float32),
    mesh=_mesh,
    scratch_types=[
        pltpu.VMEM_SHARED((NP,), jnp.float32),
        pltpu.VMEM_SHARED((NP,), jnp.float32),
        pltpu.VMEM((NP,), jnp.int32),
        pltpu.VMEM((VRP,), jnp.float32),
        [pltpu.VMEM((CHUNK,), jnp.int32) for _ in range(2)],
        [pltpu.VMEM((CHUNK,), jnp.int32) for _ in range(2)],
        [pltpu.VMEM((CHUNK,), jnp.float32) for _ in range(2)],
        [pltpu.VMEM((DSLAB,), jnp.float32) for _ in range(2)],
        [pltpu.VMEM((CHUNK,), jnp.float32) for _ in range(2)],
        [pltpu.VMEM((CHUNK,), jnp.float32) for _ in range(2)],
        [pltpu.SemaphoreType.DMA for _ in range(2)],
        [pltpu.SemaphoreType.DMA for _ in range(2)],
        [pltpu.SemaphoreType.DMA for _ in range(2)],
    ],
    compiler_params=pltpu.CompilerParams(
        needs_layout_passes=False, use_tc_tiling_on_sc=False
    ),
)
def _sc_dpdt(ei_hbm, ej_hbm, vr_hbm, upk_hbm, dblk_hbm, rad_hbm,
             out_hbm,
             vrt, dpdt, upk, vrbuf,
             ei, ej, rad, dst, vrj, cbuf, slin, sgat, ssc):
    cid = lax.axis_index("c")
    sid = lax.axis_index("s")
    wid = sid * NC + cid
    iota = lax.iota(jnp.int32, L)
    himask = jnp.full((L,), -0x10000, jnp.int32)

    nbase = sid * RPT
    pltpu.sync_copy(upk_hbm, upk)

    def vrpiece(rows0, nrows):
        pltpu.sync_copy(vr_hbm.at[pl.ds(rows0, nrows)],
                        vrbuf.at[pl.ds(0, nrows)])
        pltpu.sync_copy(vrbuf.at[pl.ds(0, nrows)], vrt.at[pl.ds(rows0, nrows)])

    def vr_full(p, carry):
        vrpiece(nbase + p * VRP, VRP)
        return carry

    @pl.when(sid < NS - 1)
    def _():
        lax.fori_loop(0, 8, vr_full, 0)

    @pl.when(sid == NS - 1)
    def _():
        lax.fori_loop(0, 7, vr_full, 0)
        vrpiece(nbase + 7 * VRP, VRL)

    def zstep(k, carry):
        vrbuf[pl.ds(k * L, L)] = jnp.zeros((L,), jnp.float32)
        return carry

    lax.fori_loop(0, VRP // L, zstep, 0)

    def zpiece(p, carry):
        pltpu.sync_copy(vrbuf, dpdt.at[pl.ds(nbase + p * VRP, VRP)])
        return carry

    lax.fori_loop(0, 8, zpiece, 0)

    plsc.subcore_barrier()

    ebase0 = wid * EPW

    def slab_start(eb):
        return jnp.minimum(eb >> 7, N_EDGE // B - DBLKS)

    def issue_lin(eb, b):
        pltpu.async_copy(ei_hbm.at[pl.ds(eb, CHUNK)], ei[b], slin[b])
        pltpu.async_copy(ej_hbm.at[pl.ds(eb, CHUNK)], ej[b], slin[b])
        pltpu.async_copy(rad_hbm.at[pl.ds(eb, CHUNK)], rad[b], slin[b])
        pltpu.async_copy(dblk_hbm.at[pl.ds(slab_start(eb) * (2 * B), DSLAB)],
                         dst[b], slin[b])

    def wait_lin(b):
        pltpu.make_async_copy(ei_hbm.at[pl.ds(0, CHUNK)], ei[b], slin[b]).wait()
        pltpu.make_async_copy(ej_hbm.at[pl.ds(0, CHUNK)], ej[b], slin[b]).wait()
        pltpu.make_async_copy(rad_hbm.at[pl.ds(0, CHUNK)], rad[b],
                              slin[b]).wait()
        pltpu.make_async_copy(dblk_hbm.at[pl.ds(0, DSLAB)], dst[b],
                              slin[b]).wait()

    def compute(eb, b, k0, ksteps):
        off0 = eb - slab_start(eb) * B

        def step(k, inner):
            sl = pl.ds(k * L, L)
            r = rad[b][sl]
            q = jnp.minimum(jnp.maximum(r, 0.0), 1.0)
            om = 1.0 - q
            w = (om * om) * (om * q) * KGRAD
            off = off0 + k * L + iota
            idx = ((off >> 7) << 8) + (off & (B - 1))
            dx = plsc.load_gather(dst[b], [idx])
            dy = plsc.load_gather(dst[b], [idx + B])
            wj = plsc.load_gather(upk, [ej[b][sl]])
            wi = plsc.load_gather(upk, [ei[b][sl]])
            uxj = plsc.bitcast(wj << 16, jnp.float32)
            uyj = plsc.bitcast(wj & himask, jnp.float32)
            uxi = plsc.bitcast(wi << 16, jnp.float32)
            uyi = plsc.bitcast(wi & himask, jnp.float32)
            prod = (uxj - uxi) * dx + (uyj - uyi) * dy
            cbuf[b][sl] = prod * w * vrj[b][sl]
            return inner

        lax.fori_loop(k0, k0 + ksteps, step, 0)

    def wait_scatter(b):
        pltpu.make_async_copy(cbuf[b], dpdt.at[ei[b]], ssc[b]).wait()

    issue_lin(ebase0, 0)
    CH2 = CHUNK // 2

    def half(eb, b, skip_scwait, issue_next):
        wait_lin(b)
        ga = pltpu.async_copy(vrt.at[ej[b].at[pl.ds(0, CH2)]],
                              vrj[b].at[pl.ds(0, CH2)], sgat[b])
        gb = pltpu.async_copy(vrt.at[ej[b].at[pl.ds(CH2, CH2)]],
                              vrj[b].at[pl.ds(CH2, CH2)], sgat[b])

        @pl.when(jnp.logical_not(skip_scwait))
        def _():
            wait_scatter(1 - b)

        @pl.when(issue_next)
        def _():
            issue_lin(eb + CHUNK, 1 - b)

        ga.wait()
        compute(eb, b, 0, CSTEPS // 2)
        gb.wait()
        compute(eb, b, CSTEPS // 2, CSTEPS // 2)
        pltpu.async_copy(cbuf[b], dpdt.at[ei[b]], ssc[b], add=True)

    true_s = jnp.bool_(True)

    def pair_body(c2, carry):
        eb0 = ebase0 + (2 * c2) * CHUNK
        half(eb0, 0, c2 == 0, true_s)
        half(eb0 + CHUNK, 1, jnp.bool_(False), c2 < NPAIR - 1)
        return carry

    lax.fori_loop(0, NPAIR, pair_body, 0)
    wait_scatter(1)

    plsc.subcore_barrier()
    pltpu.sync_copy(dpdt.at[pl.ds(nbase, RPT)],
                    out_hbm.at[cid, pl.ds(nbase, RPT)])


def _combine_body(p_ref, o_ref):
    o_ref[...] = -(p_ref[0, :N_PART] + p_ref[1, :N_PART])


_combine = pl.pallas_call(
    _combine_body,
    out_shape=jax.ShapeDtypeStruct((N_PART,), jnp.float32),
)


def _prep_body(u_ref, v_ref, rho_ref, upk_ref, vr_ref):
    x = jax.lax.bitcast_convert_type(u_ref[:, 0, :], jnp.int32)
    y = jax.lax.bitcast_convert_type(u_ref[:, 1, :], jnp.int32)
    lo = lax.shift_right_logical(x + 0x8000, 16)
    hi = (y + 0x8000) & jnp.int32(-0x10000)
    upk_ref[...] = lo | hi
    vr_ref[...] = v_ref[...] * rho_ref[...]


_prep = pl.pallas_call(
    _prep_body,
    out_shape=(
        jax.ShapeDtypeStruct((NP // B, B), jnp.int32),
        jax.ShapeDtypeStruct((N_PART,), jnp.float32),
    ),
)


def _planar(x, nblocks):
    return x.reshape(nblocks, B, 2).transpose(0, 2, 1)


def kernel(edge_i, edge_j, V, rho, u, distances, radialDistances):
    ei = edge_i.astype(jnp.int32)
    ej = edge_j.astype(jnp.int32)
    u_p = jnp.pad(u, ((0, NP - N_PART), (0, 0)))
    upkd, vr = _prep(_planar(u_p, NP // B), V, rho)
    dblk = _planar(distances, N_EDGE // B).reshape(-1)
    part = _sc_dpdt(ei, ej, vr, upkd.reshape(-1), dblk, radialDistances)
    return _combine(part)
